# baseline (speedup 1.0000x reference)
import functools

import jax
import jax.numpy as jnp
from jax import lax
from jax.experimental import pallas as pl
from jax.experimental.pallas import tpu as pltpu

NC = 2
NPR = 3


def _fused_body(x_ref, m_ref, lp_ref, lt_ref, tr_ref, c0_ref, c1_ref, tg_ref,
                out_ref, acc_ref, *, grid_i, grid_j, A, gts_den):
    i = pl.program_id(0)
    j = pl.program_id(1)
    step = i * grid_j + j
    last = grid_i * grid_j - 1

    @pl.when(step == 0)
    def _init():
        acc_ref[0] = 0.0
        acc_ref[1] = 0.0

    x = x_ref[...]
    m = m_ref[...]
    cls_pat = lax.broadcasted_iota(jnp.int32, x.shape, len(x.shape) - 1) & 1
    g = (m == cls_pat)
    bce = jnp.maximum(x, 0.0) + jnp.log1p(jnp.exp(-jnp.abs(x)))
    part = jnp.sum(bce) - jnp.sum(jnp.where(g, x, 0.0))
    acc_ref[0] = acc_ref[0] + part

    d = lp_ref[...] - lt_ref[...]
    ad = jnp.abs(d)
    sl1 = jnp.where(ad < 1.0, 0.5 * d * d, ad - 0.5)
    posl = jnp.clip(tr_ref[...], 0, 1) > 0
    acc_ref[1] = acc_ref[1] + jnp.sum(jnp.where(posl, sl1, 0.0))

    @pl.when(step == last)
    def _cls():
        c0 = c0_ref[...]
        c1 = c1_ref[...]
        t = jnp.clip(tg_ref[...], 0, 1)
        pos = t > 0
        dmax = jnp.maximum(c0, c1)
        gathered = jnp.where(pos, c1, c0)
        ce = dmax - gathered + jnp.log1p(jnp.exp(-jnp.abs(c0 - c1)))
        loss = jnp.where(pos, 0.0, ce)

        posf = pos.astype(jnp.float32)
        num_pos = jnp.sum(posf, axis=1, keepdims=True)
        num_neg = jnp.minimum(NPR * num_pos, float(A - 1))

        def body(_, carry):
            lo, hi = carry
            mid = lo + lax.shift_right_logical(hi - lo, 1)
            thr = lax.bitcast_convert_type(mid, jnp.float32)
            cnt = jnp.sum((loss > thr).astype(jnp.float32), axis=1,
                          keepdims=True)
            pred = cnt < num_neg
            return (jnp.where(pred, lo, mid + 1), jnp.where(pred, mid, hi))

        lo0 = jnp.zeros(num_pos.shape, jnp.int32)
        hi0 = jnp.full(num_pos.shape, 0x7F800000, jnp.int32)
        lo, _ = lax.fori_loop(0, 31, body, (lo0, hi0))
        vstar = lax.bitcast_convert_type(lo, jnp.float32)

        gt_mask = loss > vstar
        big = jnp.sum((gt_mask).astype(jnp.float32), axis=1, keepdims=True)
        tie = jnp.where(num_neg > 0, vstar * (num_neg - big), 0.0)
        cls_row = (jnp.sum(jnp.where(pos, ce, 0.0), axis=1, keepdims=True)
                   + jnp.sum(jnp.where(gt_mask, ce, 0.0), axis=1,
                             keepdims=True)
                   + tie)

        n_tot = jnp.sum(num_pos)
        loc_loss = acc_ref[1] / n_tot
        cls_loss = jnp.sum(cls_row) / n_tot
        lane = lax.broadcasted_iota(jnp.int32, (1, 128), 1)
        vec = jnp.where(lane == 0, loc_loss,
                        jnp.where(lane == 1, cls_loss,
                                  jnp.where(lane == 2, acc_ref[0] / gts_den,
                                            0.0)))
        out_ref[...] = vec


def kernel(loc_preds, loc_targets, cls_preds, cls_targets, global_text_segs,
           gts_masks):
    B, A, K = loc_preds.shape
    L = global_text_segs.shape[0]
    H, W = gts_masks.shape[1:]
    WC = W * NC

    x_gts = global_text_segs.reshape(L, B, H, WC)
    mrep = jnp.repeat(gts_masks, NC, axis=-1)
    steps_total = 16
    lwb = (A * K * B) // steps_total
    lp = loc_preds.reshape(steps_total, 1, lwb)
    lt = loc_targets.reshape(steps_total, 1, lwb)
    trep = jnp.repeat(cls_targets, K, axis=-1).reshape(steps_total, 1, lwb)
    c0 = cls_preds[:, :, 0]
    c1 = cls_preds[:, :, 1]

    GI, GJ = 4, 4
    BB = B // GI
    assert GI * GJ == steps_total

    in_specs = [
            pl.BlockSpec((1, BB, H, WC), lambda i, j: (j, i, 0, 0)),
            pl.BlockSpec((BB, H, WC), lambda i, j: (i, 0, 0)),
            pl.BlockSpec((1, 1, lwb), lambda i, j: (i * GJ + j, 0, 0)),
            pl.BlockSpec((1, 1, lwb), lambda i, j: (i * GJ + j, 0, 0)),
            pl.BlockSpec((1, 1, lwb), lambda i, j: (i * GJ + j, 0, 0)),
            pl.BlockSpec((B, A), lambda i, j: (0, 0)),
            pl.BlockSpec((B, A), lambda i, j: (0, 0)),
            pl.BlockSpec((B, A), lambda i, j: (0, 0)),
    ]

    body = functools.partial(_fused_body, grid_i=GI, grid_j=GJ, A=A,
                             gts_den=float(L * B * H * W * NC))
    out = pl.pallas_call(
        body,
        grid=(GI, GJ),
        in_specs=in_specs,
        out_specs=pl.BlockSpec((1, 128), lambda i, j: (0, 0)),
        out_shape=jax.ShapeDtypeStruct((1, 128), jnp.float32),
        scratch_shapes=[pltpu.SMEM((2,), jnp.float32)],
        compiler_params=pltpu.CompilerParams(
            dimension_semantics=("arbitrary", "arbitrary")),
    )(x_gts, mrep, lp, lt, trep, c0, c1, cls_targets)

    return (out[0, 0], out[0, 1], out[0, 2])


# speedup vs baseline: 1.0402x; 1.0402x over previous
import functools

import jax
import jax.numpy as jnp
from jax import lax
from jax.experimental import pallas as pl
from jax.experimental.pallas import tpu as pltpu

NC = 2
NPR = 3
_HI = jax.lax.Precision.HIGHEST


def _even_odd_mats(pair_w):
    i = lax.broadcasted_iota(jnp.int32, (2 * pair_w, pair_w), 0)
    j = lax.broadcasted_iota(jnp.int32, (2 * pair_w, pair_w), 1)
    e0 = (i == 2 * j).astype(jnp.float32)
    e1 = (i == 2 * j + 1).astype(jnp.float32)
    return e0, e1


def _fused_body(x_ref, m_ref, lp_ref, lt_ref, t8_ref, c0_ref, c1_ref, tg_ref,
                out_ref, acc_ref, *, grid_i, grid_j, A, gts_den):
    i = pl.program_id(0)
    j = pl.program_id(1)
    step = i * grid_j + j
    last = grid_i * grid_j - 1

    @pl.when(step == 0)
    def _init():
        acc_ref[0] = 0.0
        acc_ref[1] = 0.0

    x = x_ref[...]
    bce_sp = jnp.sum(jnp.maximum(x, 0.0) + jnp.log1p(jnp.exp(-jnp.abs(x))))
    xr = x.reshape(1024, 256)
    e0, e1 = _even_odd_mats(128)
    x_c0 = jnp.dot(xr, e0, precision=_HI)
    x_c1 = jnp.dot(xr, e1, precision=_HI)
    mf = (m_ref[...].reshape(1024, 128) > 0).astype(jnp.float32)
    gathered = jnp.sum(x_c0 + mf * (x_c1 - x_c0))
    acc_ref[0] = acc_ref[0] + (bce_sp - gathered)

    d = lp_ref[...] - lt_ref[...]
    ad = jnp.abs(d)
    sl1 = jnp.where(ad < 1.0, 0.5 * d * d, ad - 0.5).reshape(512, 128)
    tpos = (jnp.clip(t8_ref[...], 0, 1) > 0).astype(jnp.float32)
    t64 = tpos.reshape(64, 128)
    io = lax.broadcasted_iota(jnp.int32, (128, 1024), 0)
    jo = lax.broadcasted_iota(jnp.int32, (128, 1024), 1)
    rmat = (io == (jo >> 3)).astype(jnp.float32)
    posrep = jnp.dot(t64, rmat, precision=_HI).reshape(512, 128)
    acc_ref[1] = acc_ref[1] + jnp.sum(sl1 * posrep)

    @pl.when(step == last)
    def _cls():
        c0 = c0_ref[...]
        c1 = c1_ref[...]
        t = jnp.clip(tg_ref[...], 0, 1)
        pos = t > 0
        dmax = jnp.maximum(c0, c1)
        gat = jnp.where(pos, c1, c0)
        ce = dmax - gat + jnp.log1p(jnp.exp(-jnp.abs(c0 - c1)))
        loss = jnp.where(pos, 0.0, ce)

        posf = pos.astype(jnp.float32)
        num_pos = jnp.sum(posf, axis=1, keepdims=True)
        num_neg = jnp.minimum(NPR * num_pos, float(A - 1))

        def body(_, carry):
            lo, hi = carry
            mid = lo + lax.shift_right_logical(hi - lo, 1)
            thr = lax.bitcast_convert_type(mid, jnp.float32)
            cnt = jnp.sum((loss > thr).astype(jnp.float32), axis=1,
                          keepdims=True)
            pred = cnt < num_neg
            return (jnp.where(pred, lo, mid + 1), jnp.where(pred, mid, hi))

        lo0 = jnp.zeros(num_pos.shape, jnp.int32)
        hi0 = jnp.full(num_pos.shape, 0x7F800000, jnp.int32)
        lo, _ = lax.fori_loop(0, 31, body, (lo0, hi0))
        vstar = lax.bitcast_convert_type(lo, jnp.float32)

        gt_mask = loss > vstar
        big = jnp.sum(gt_mask.astype(jnp.float32), axis=1, keepdims=True)
        tie = jnp.where(num_neg > 0, vstar * (num_neg - big), 0.0)
        cls_row = (jnp.sum(jnp.where(pos, ce, 0.0), axis=1, keepdims=True)
                   + jnp.sum(jnp.where(gt_mask, ce, 0.0), axis=1,
                             keepdims=True)
                   + tie)

        n_tot = jnp.sum(num_pos)
        loc_loss = acc_ref[1] / n_tot
        cls_loss = jnp.sum(cls_row) / n_tot
        lane = lax.broadcasted_iota(jnp.int32, (1, 128), 1)
        vec = jnp.where(lane == 0, loc_loss,
                        jnp.where(lane == 1, cls_loss,
                                  jnp.where(lane == 2, acc_ref[0] / gts_den,
                                            0.0)))
        out_ref[...] = vec


def kernel(loc_preds, loc_targets, cls_preds, cls_targets, global_text_segs,
           gts_masks):
    B, A, K = loc_preds.shape
    L = global_text_segs.shape[0]
    H, W = gts_masks.shape[1:]
    WC = W * NC

    x_gts = global_text_segs.reshape(L, B, H, WC)
    lp = loc_preds.reshape(B, A * K)
    lt = loc_targets.reshape(B, A * K)
    c0 = cls_preds[:, :, 0]
    c1 = cls_preds[:, :, 1]

    GI, GJ = 4, 4
    BB = B // GI
    lwb = (A * K) // (GI * GJ)
    awb = A // (GI * GJ)

    in_specs = [
        pl.BlockSpec((1, BB, H, WC), lambda i, j: (j, i, 0, 0)),
        pl.BlockSpec((BB, H, W), lambda i, j: (i, 0, 0)),
        pl.BlockSpec((B, lwb), lambda i, j: (0, i * GJ + j)),
        pl.BlockSpec((B, lwb), lambda i, j: (0, i * GJ + j)),
        pl.BlockSpec((B, awb), lambda i, j: (0, i * GJ + j)),
        pl.BlockSpec((B, A), lambda i, j: (0, 0)),
        pl.BlockSpec((B, A), lambda i, j: (0, 0)),
        pl.BlockSpec((B, A), lambda i, j: (0, 0)),
    ]

    body = functools.partial(_fused_body, grid_i=GI, grid_j=GJ, A=A,
                             gts_den=float(L * B * H * W * NC))
    out = pl.pallas_call(
        body,
        grid=(GI, GJ),
        in_specs=in_specs,
        out_specs=pl.BlockSpec((1, 128), lambda i, j: (0, 0)),
        out_shape=jax.ShapeDtypeStruct((1, 128), jnp.float32),
        scratch_shapes=[pltpu.SMEM((2,), jnp.float32)],
        compiler_params=pltpu.CompilerParams(
            dimension_semantics=("arbitrary", "arbitrary")),
    )(x_gts, gts_masks, lp, lt, cls_targets, c0, c1, cls_targets)

    return (out[0, 0], out[0, 1], out[0, 2])


# speedup vs baseline: 8.6194x; 8.2866x over previous
import functools

import jax
import jax.numpy as jnp
from jax import lax
from jax.experimental import pallas as pl
from jax.experimental.pallas import tpu as pltpu

NC = 2
NPR = 3


def _fused_body(x_ref, m_ref, lp_ref, lt_ref, t8_ref, cp_ref, tg_ref,
                out_ref, acc_ref, *, grid_i, grid_j, A, gts_den):
    i = pl.program_id(0)
    j = pl.program_id(1)
    step = i * grid_j + j
    last = grid_i * grid_j - 1

    @pl.when(step == 0)
    def _init():
        acc_ref[0] = 0.0
        acc_ref[1] = 0.0

    x = x_ref[...].reshape(2048, 128)
    bce_sp = jnp.sum(jnp.maximum(x, 0.0) + jnp.log1p(jnp.exp(-jnp.abs(x))))
    xp = x.reshape(1024, 256)
    x_c0 = xp[:, :128]
    x_c1 = xp[:, 128:]
    mf = m_ref[...].reshape(1024, 128) > 0
    gathered = jnp.sum(jnp.where(mf, x_c1, x_c0))
    acc_ref[0] = acc_ref[0] + (bce_sp - gathered)

    d = lp_ref[...] - lt_ref[...]
    ad = jnp.abs(d)
    sl1 = jnp.where(ad < 1.0, 0.5 * d * d, ad - 0.5)
    posl = (jnp.clip(t8_ref[...], 0, 1) > 0)[:, None, :]
    acc_ref[1] = acc_ref[1] + jnp.sum(jnp.where(posl, sl1, 0.0))

    @pl.when(step == last)
    def _cls():
        cp = cp_ref[...].reshape(1024, 256)
        c0 = cp[:, :128].reshape(8, 128, 128)
        c1 = cp[:, 128:].reshape(8, 128, 128)
        t = jnp.clip(tg_ref[...].reshape(8, 128, 128), 0, 1)
        pos = t > 0
        dmax = jnp.maximum(c0, c1)
        gat = jnp.where(pos, c1, c0)
        ce = dmax - gat + jnp.log1p(jnp.exp(-jnp.abs(c0 - c1)))
        loss = jnp.where(pos, 0.0, ce)

        def rsum(v):
            return jnp.sum(jnp.sum(v, axis=2, keepdims=True), axis=1,
                           keepdims=True)

        posf = pos.astype(jnp.float32)
        num_pos = rsum(posf)
        num_neg = jnp.minimum(NPR * num_pos, float(A - 1))

        def body(_, carry):
            lo, hi = carry
            mid = lo + lax.shift_right_logical(hi - lo, 1)
            thr = lax.bitcast_convert_type(mid, jnp.float32)
            cnt = rsum((loss > thr).astype(jnp.float32))
            pred = cnt < num_neg
            return (jnp.where(pred, lo, mid + 1), jnp.where(pred, mid, hi))

        lo0 = jnp.zeros(num_pos.shape, jnp.int32)
        hi0 = jnp.full(num_pos.shape, 0x7F800000, jnp.int32)
        lo, _ = lax.fori_loop(0, 31, body, (lo0, hi0))
        vstar = lax.bitcast_convert_type(lo, jnp.float32)

        gt_mask = loss > vstar
        big = rsum(gt_mask.astype(jnp.float32))
        tie = jnp.where(num_neg > 0, vstar * (num_neg - big), 0.0)
        cls_row = (rsum(jnp.where(pos, ce, 0.0))
                   + rsum(jnp.where(gt_mask, ce, 0.0)) + tie)

        n_tot = jnp.sum(num_pos)
        loc_loss = acc_ref[1] / n_tot
        cls_loss = jnp.sum(cls_row) / n_tot
        lane = lax.broadcasted_iota(jnp.int32, (1, 128), 1)
        vec = jnp.where(lane == 0, loc_loss,
                        jnp.where(lane == 1, cls_loss,
                                  jnp.where(lane == 2, acc_ref[0] / gts_den,
                                            0.0)))
        out_ref[...] = vec


def kernel(loc_preds, loc_targets, cls_preds, cls_targets, global_text_segs,
           gts_masks):
    B, A, K = loc_preds.shape
    L = global_text_segs.shape[0]
    H, W = gts_masks.shape[1:]

    lp = jnp.transpose(loc_preds, (0, 2, 1))
    lt = jnp.transpose(loc_targets, (0, 2, 1))
    cp = cls_preds.reshape(B, A // 128, 128, NC).transpose(0, 1, 3, 2)
    cp = cp.reshape(B, (A // 128) * NC, 128)
    xg = global_text_segs.reshape(L, B, H, W // 128, 128, NC)
    xg = xg.transpose(0, 1, 2, 3, 5, 4).reshape(L, B, H * (W // 128) * NC, 128)

    GI, GJ = 4, 4
    BB = B // GI
    steps = GI * GJ
    awb = A // steps

    in_specs = [
        pl.BlockSpec((1, BB, H * (W // 128) * NC, 128),
                     lambda i, j: (j, i, 0, 0)),
        pl.BlockSpec((BB, H, W), lambda i, j: (i, 0, 0)),
        pl.BlockSpec((B, K, awb), lambda i, j: (0, 0, i * GJ + j)),
        pl.BlockSpec((B, K, awb), lambda i, j: (0, 0, i * GJ + j)),
        pl.BlockSpec((B, awb), lambda i, j: (0, i * GJ + j)),
        pl.BlockSpec((B, (A // 128) * NC, 128), lambda i, j: (0, 0, 0)),
        pl.BlockSpec((B, A), lambda i, j: (0, 0)),
    ]

    body = functools.partial(_fused_body, grid_i=GI, grid_j=GJ, A=A,
                             gts_den=float(L * B * H * W * NC))
    out = pl.pallas_call(
        body,
        grid=(GI, GJ),
        in_specs=in_specs,
        out_specs=pl.BlockSpec((1, 128), lambda i, j: (0, 0)),
        out_shape=jax.ShapeDtypeStruct((1, 128), jnp.float32),
        scratch_shapes=[pltpu.SMEM((2,), jnp.float32)],
        compiler_params=pltpu.CompilerParams(
            dimension_semantics=("arbitrary", "arbitrary")),
    )(xg, gts_masks, lp, lt, cls_targets, cp, cls_targets)

    return (out[0, 0], out[0, 1], out[0, 2])
